# baseline (speedup 1.0000x reference)
import jax
import jax.numpy as jnp
from jax.experimental import pallas as pl
from jax.experimental.pallas import tpu as pltpu

_K = 8
_ROWS = 128
_COLS = 32768
_LANES = 128
_STEPS = _COLS // _LANES
_R = 32


def _topk_body(x_ref, o_ref):
    init = tuple(jnp.full((_R, _LANES), -jnp.inf, jnp.float32) for _ in range(_K))

    def step(i, ts):
        cur = x_ref[:, pl.ds(i * _LANES, _LANES)]
        new = []
        for t in ts:
            hi = jnp.maximum(t, cur)
            cur = jnp.minimum(t, cur)
            new.append(hi)
        return tuple(new)

    ts = jax.lax.fori_loop(0, _STEPS, step, init)
    cand = jnp.concatenate(ts, axis=1)

    n = _K * _LANES
    iota = jax.lax.broadcasted_iota(jnp.int32, (_R, n), 1)
    big = jnp.int32(2**30)
    outs = []
    c = cand
    for _ in range(_K):
        m = jnp.max(c, axis=1, keepdims=True)
        outs.append(m)
        idx = jnp.min(jnp.where(c == m, iota, big), axis=1, keepdims=True)
        c = jnp.where(iota == idx, -jnp.inf, c)
    o_ref[...] = jnp.concatenate(outs, axis=1)


def kernel(inputs):
    grid = _ROWS // _R
    return pl.pallas_call(
        _topk_body,
        grid=(grid,),
        in_specs=[pl.BlockSpec((_R, _COLS), lambda i: (i, 0))],
        out_specs=pl.BlockSpec((_R, _K), lambda i: (i, 0)),
        out_shape=jax.ShapeDtypeStruct((_ROWS, _K), jnp.float32),
    )(inputs)


# speedup vs baseline: 1.1245x; 1.1245x over previous
import jax
import jax.numpy as jnp
from jax.experimental import pallas as pl

_K = 8
_ROWS = 128
_COLS = 32768
_LANES = 128
_STEPS = _COLS // _LANES
_GROUPS = _STEPS // _K
_R = 16

_SORT8 = ((0, 1), (2, 3), (4, 5), (6, 7),
          (0, 2), (1, 3), (4, 6), (5, 7),
          (1, 2), (5, 6),
          (0, 4), (1, 5), (2, 6), (3, 7),
          (2, 4), (3, 5),
          (1, 2), (3, 4), (5, 6))
_MERGE8 = ((0, 4), (1, 5), (2, 6), (3, 7),
           (0, 2), (1, 3), (4, 6), (5, 7),
           (0, 1), (2, 3), (4, 5), (6, 7))


def _apply_net(net, a):
    for i, j in net:
        hi = jnp.maximum(a[i], a[j])
        lo = jnp.minimum(a[i], a[j])
        a[i], a[j] = hi, lo
    return a


def _topk_body(x_ref, o_ref):
    init = tuple(jnp.full((_R, _LANES), -jnp.inf, jnp.float32) for _ in range(_K))

    def group(g, ts):
        base = g * (_K * _LANES)
        n = [x_ref[:, pl.ds(base + k * _LANES, _LANES)] for k in range(_K)]
        n = _apply_net(_SORT8, n)
        m = [jnp.maximum(ts[i], n[_K - 1 - i]) for i in range(_K)]
        m = _apply_net(_MERGE8, m)
        return tuple(m)

    ts = jax.lax.fori_loop(0, _GROUPS, group, init)
    cand = jnp.concatenate(ts, axis=1)

    nw = _K * _LANES
    iota = jax.lax.broadcasted_iota(jnp.int32, (_R, nw), 1)
    big = jnp.int32(2**30)
    outs = []
    c = cand
    for _ in range(_K):
        mx = jnp.max(c, axis=1, keepdims=True)
        outs.append(mx)
        idx = jnp.min(jnp.where(c == mx, iota, big), axis=1, keepdims=True)
        c = jnp.where(iota == idx, -jnp.inf, c)
    o_ref[...] = jnp.concatenate(outs, axis=1)


def kernel(inputs):
    grid = _ROWS // _R
    return pl.pallas_call(
        _topk_body,
        grid=(grid,),
        in_specs=[pl.BlockSpec((_R, _COLS), lambda i: (i, 0))],
        out_specs=pl.BlockSpec((_R, _K), lambda i: (i, 0)),
        out_shape=jax.ShapeDtypeStruct((_ROWS, _K), jnp.float32),
    )(inputs)


# speedup vs baseline: 1.5173x; 1.3493x over previous
import jax
import jax.numpy as jnp
from jax.experimental import pallas as pl

_K = 8
_ROWS = 128
_COLS = 32768
_LANES = 128
_STEPS = _COLS // _LANES
_GROUPS = _STEPS // _K
_R = 32

_SORT8 = ((0, 1), (2, 3), (4, 5), (6, 7),
          (0, 2), (1, 3), (4, 6), (5, 7),
          (1, 2), (5, 6),
          (0, 4), (1, 5), (2, 6), (3, 7),
          (2, 4), (3, 5),
          (1, 2), (3, 4), (5, 6))
_MERGE8 = ((0, 4), (1, 5), (2, 6), (3, 7),
           (0, 2), (1, 3), (4, 6), (5, 7),
           (0, 1), (2, 3), (4, 5), (6, 7))


def _apply_net(net, a):
    for i, j in net:
        hi = jnp.maximum(a[i], a[j])
        lo = jnp.minimum(a[i], a[j])
        a[i], a[j] = hi, lo
    return a


def _topk_body(x_ref, o_ref):
    init = tuple(jnp.full((_R, _LANES), -jnp.inf, jnp.float32) for _ in range(_K))

    def group(g, ts):
        base = g * (_K * _LANES)
        n = [x_ref[:, pl.ds(base + k * _LANES, _LANES)] for k in range(_K)]
        n = _apply_net(_SORT8, n)
        m = [jnp.maximum(ts[i], n[_K - 1 - i]) for i in range(_K)]
        m = _apply_net(_MERGE8, m)
        return tuple(m)

    ts = jax.lax.fori_loop(0, _GROUPS, group, init)
    cand = jnp.concatenate(ts, axis=1)

    nw = _K * _LANES
    iota = jax.lax.broadcasted_iota(jnp.int32, (_R, nw), 1)
    big = jnp.int32(2**30)
    outs = []
    c = cand
    for _ in range(_K):
        mx = jnp.max(c, axis=1, keepdims=True)
        outs.append(mx)
        idx = jnp.min(jnp.where(c == mx, iota, big), axis=1, keepdims=True)
        c = jnp.where(iota == idx, -jnp.inf, c)
    o_ref[...] = jnp.concatenate(outs, axis=1)


def kernel(inputs):
    grid = _ROWS // _R
    return pl.pallas_call(
        _topk_body,
        grid=(grid,),
        in_specs=[pl.BlockSpec((_R, _COLS), lambda i: (i, 0))],
        out_specs=pl.BlockSpec((_R, _K), lambda i: (i, 0)),
        out_shape=jax.ShapeDtypeStruct((_ROWS, _K), jnp.float32),
    )(inputs)


# speedup vs baseline: 1.6093x; 1.0606x over previous
import jax
import jax.numpy as jnp
from jax.experimental import pallas as pl

_K = 8
_ROWS = 128
_COLS = 32768
_LANES = 128
_STEPS = _COLS // _LANES
_GROUPS = _STEPS // _K
_R = 64

_SORT8 = ((0, 1), (2, 3), (4, 5), (6, 7),
          (0, 2), (1, 3), (4, 6), (5, 7),
          (1, 2), (5, 6),
          (0, 4), (1, 5), (2, 6), (3, 7),
          (2, 4), (3, 5),
          (1, 2), (3, 4), (5, 6))
_MERGE8 = ((0, 4), (1, 5), (2, 6), (3, 7),
           (0, 2), (1, 3), (4, 6), (5, 7),
           (0, 1), (2, 3), (4, 5), (6, 7))


def _apply_net(net, a):
    for i, j in net:
        hi = jnp.maximum(a[i], a[j])
        lo = jnp.minimum(a[i], a[j])
        a[i], a[j] = hi, lo
    return a


def _topk_body(x_ref, o_ref):
    init = tuple(jnp.full((_R, _LANES), -jnp.inf, jnp.float32) for _ in range(_K))

    def group(g, ts):
        base = g * (_K * _LANES)
        n = [x_ref[:, pl.ds(base + k * _LANES, _LANES)] for k in range(_K)]
        n = _apply_net(_SORT8, n)
        m = [jnp.maximum(ts[i], n[_K - 1 - i]) for i in range(_K)]
        m = _apply_net(_MERGE8, m)
        return tuple(m)

    ts = jax.lax.fori_loop(0, _GROUPS, group, init)
    cand = jnp.concatenate(ts, axis=1)

    nw = _K * _LANES
    iota = jax.lax.broadcasted_iota(jnp.int32, (_R, nw), 1)
    big = jnp.int32(2**30)
    outs = []
    c = cand
    for _ in range(_K):
        mx = jnp.max(c, axis=1, keepdims=True)
        outs.append(mx)
        idx = jnp.min(jnp.where(c == mx, iota, big), axis=1, keepdims=True)
        c = jnp.where(iota == idx, -jnp.inf, c)
    o_ref[...] = jnp.concatenate(outs, axis=1)


def kernel(inputs):
    grid = _ROWS // _R
    return pl.pallas_call(
        _topk_body,
        grid=(grid,),
        in_specs=[pl.BlockSpec((_R, _COLS), lambda i: (i, 0))],
        out_specs=pl.BlockSpec((_R, _K), lambda i: (i, 0)),
        out_shape=jax.ShapeDtypeStruct((_ROWS, _K), jnp.float32),
    )(inputs)
